# Initial kernel scaffold; baseline (speedup 1.0000x reference)
import jax
import jax.numpy as jnp
from jax.experimental import pallas as pl


def kernel(x, edge_index, W_low1, W_high1, W_mlp1, av_low1, av_high1, av_mlp1, attv1, W_low2, W_high2, W_mlp2, av_low2, av_high2, av_mlp2, attv2):
    raise NotImplementedError("write your pallas kernel here")


# speedup vs baseline: 6.3749x; 6.3749x over previous
import functools

import jax
import jax.numpy as jnp
from jax import lax
from jax.experimental import pallas as pl
from jax.experimental.pallas import tpu as pltpu
from jax.experimental.pallas import tpu_sc as plsc

N = 10000
D = 128
E = 320000
NC = 2
NS = 16
NW = NC * NS
EDGES_PER_TILE = E // NS
DEG_EDGES_PER_TILE = E // NW
ZCHUNK = 80
NZ = N // ZCHUNK
ZITER = (NZ + NS - 1) // NS
SUB = 80
NSUB = 1
CHUNK = SUB * NSUB
NCHUNK = EDGES_PER_TILE // CHUNK
DEG_NCHUNK = DEG_EDGES_PER_TILE // CHUNK

_mesh = plsc.VectorSubcoreMesh(
    core_axis_name="c", subcore_axis_name="s", num_cores=NC, num_subcores=NS
)


def _make_sc_spmm(with_deg):
    outs = [
        jax.ShapeDtypeStruct((N, D), jnp.float32),
        jax.ShapeDtypeStruct((N, D), jnp.float32),
    ]
    if with_deg:
        outs += [
            jax.ShapeDtypeStruct((N, D), jnp.float32),
            jax.ShapeDtypeStruct((N, D), jnp.float32),
        ]

    def body(hl, hh, row_hbm, col_hbm, *rest):
        if with_deg:
            out_l, out_h, deg0, deg1 = rest[:4]
            rbuf, cbuf, rows, zbuf, acc, gsem = rest[4:]
        else:
            out_l, out_h = rest[:2]
            rbuf, cbuf, rows, zbuf, acc, gsem = rest[2:]

        c = lax.axis_index("c")
        s = lax.axis_index("s")

        z16 = jnp.zeros((16,), jnp.float32)

        def zrow(r, carry):
            for k in range(D // 16):
                zbuf[r, pl.ds(k * 16, 16)] = z16
            return carry
        lax.fori_loop(0, ZCHUNK, zrow, 0)

        def zero_acc():
            for k in range(ZITER):
                cid = s + NS * k
                off = pl.multiple_of(cid * ZCHUNK, ZCHUNK)

                @pl.when(cid < NZ)
                def _(off=off):
                    pltpu.sync_copy(zbuf, acc.at[pl.ds(off, ZCHUNK)])

        def copy_out(dst):
            for k in range(ZITER):
                cid = s + NS * k
                off = pl.multiple_of(cid * ZCHUNK, ZCHUNK)

                @pl.when(cid < NZ)
                def _(off=off):
                    pltpu.sync_copy(
                        acc.at[pl.ds(off, ZCHUNK)], rows.at[pl.ds(0, ZCHUNK)])
                    pltpu.sync_copy(
                        rows.at[pl.ds(0, ZCHUNK)], dst.at[pl.ds(off, ZCHUNK)])

        zero_acc()
        plsc.subcore_barrier()

        ebase = s * EDGES_PER_TILE

        def run(h_src):
            def chunk_body(i, carry):
                off = ebase + i * CHUNK
                for j in range(NSUB):
                    pltpu.sync_copy(
                        row_hbm.at[pl.ds(off + j * SUB, SUB)], rbuf.at[j])
                    pltpu.sync_copy(
                        col_hbm.at[pl.ds(off + j * SUB, SUB)], cbuf.at[j])
                descs = [
                    pltpu.async_copy(
                        h_src.at[cbuf.at[j]],
                        rows.at[pl.ds(j * SUB, SUB)],
                        gsem,
                    )
                    for j in range(NSUB)
                ]
                for d_ in descs:
                    d_.wait()
                for j in range(NSUB):
                    pltpu.sync_copy(
                        rows.at[pl.ds(j * SUB, SUB)],
                        acc.at[rbuf.at[j]],
                        add=True,
                    )
                return carry
            lax.fori_loop(0, NCHUNK, chunk_body, 0)

        @pl.when(c == 0)
        def _():
            run(hl)

        @pl.when(c == 1)
        def _():
            run(hh)

        plsc.subcore_barrier()

        @pl.when(c == 0)
        def _():
            copy_out(out_l)

        @pl.when(c == 1)
        def _():
            copy_out(out_h)

        if with_deg:
            plsc.subcore_barrier()
            zero_acc()
            one16 = jnp.full((16,), 1.0, jnp.float32)

            def orow(r, carry):
                for k in range(D // 16):
                    rows[r, pl.ds(k * 16, 16)] = one16
                return carry
            lax.fori_loop(0, SUB, orow, 0)
            plsc.subcore_barrier()

            dbase = (c * NS + s) * DEG_EDGES_PER_TILE

            def deg_body(i, carry):
                off = dbase + i * CHUNK
                pltpu.sync_copy(row_hbm.at[pl.ds(off, SUB)], rbuf.at[0])
                pltpu.sync_copy(
                    rows.at[pl.ds(0, SUB)], acc.at[rbuf.at[0]], add=True)
                return carry
            lax.fori_loop(0, DEG_NCHUNK, deg_body, 0)

            plsc.subcore_barrier()

            @pl.when(c == 0)
            def _():
                copy_out(deg0)

            @pl.when(c == 1)
            def _():
                copy_out(deg1)

    return pl.kernel(
        body,
        out_type=outs,
        mesh=_mesh,
        scratch_types=[
            pltpu.VMEM((NSUB, SUB), jnp.int32),
            pltpu.VMEM((NSUB, SUB), jnp.int32),
            pltpu.VMEM((CHUNK, D), jnp.float32),
            pltpu.VMEM((ZCHUNK, D), jnp.float32),
            pltpu.VMEM_SHARED((N, D), jnp.float32),
            pltpu.SemaphoreType.DMA,
        ],
        name="sc_spmm_deg" if with_deg else "sc_spmm",
    )


_sc_spmm_deg = _make_sc_spmm(True)
_sc_spmm = _make_sc_spmm(False)


_BLK = 1000
_GRID = N // _BLK


def _mm3_body(x_ref, wl_ref, wh_ref, wm_ref, hl_ref, hh_ref, hm_ref):
    xb = x_ref[...]
    hl_ref[...] = jnp.dot(xb, wl_ref[...], preferred_element_type=jnp.float32)
    hh_ref[...] = jnp.dot(xb, wh_ref[...], preferred_element_type=jnp.float32)
    hm_ref[...] = jnp.dot(xb, wm_ref[...], preferred_element_type=jnp.float32)


def _tc_matmul3(x, wl, wh, wm):
    blk = pl.BlockSpec((_BLK, D), lambda i: (i, 0))
    wspec = pl.BlockSpec((D, D), lambda i: (0, 0))
    return pl.pallas_call(
        _mm3_body,
        grid=(_GRID,),
        in_specs=[blk, wspec, wspec, wspec],
        out_specs=[blk, blk, blk],
        out_shape=[jax.ShapeDtypeStruct((N, D), jnp.float32)] * 3,
    )(x, wl, wh, wm)


def _fuse_core(sl, sh, hh, hm, inv, avl, avh, avm, attv_ref):
    ol = jnp.maximum(sl * inv, 0.0)
    oh = jnp.maximum(hh - sh * inv, 0.0)
    om = jnp.maximum(hm, 0.0)
    z0 = jax.nn.sigmoid(jnp.sum(ol * avl, axis=1, keepdims=True))
    z1 = jax.nn.sigmoid(jnp.sum(oh * avh, axis=1, keepdims=True))
    z2 = jax.nn.sigmoid(jnp.sum(om * avm, axis=1, keepdims=True))
    t = 1.0 / 3.0
    l0 = (z0 * attv_ref[0, 0] + z1 * attv_ref[1, 0] + z2 * attv_ref[2, 0]) * t
    l1 = (z0 * attv_ref[0, 1] + z1 * attv_ref[1, 1] + z2 * attv_ref[2, 1]) * t
    l2 = (z0 * attv_ref[0, 2] + z1 * attv_ref[1, 2] + z2 * attv_ref[2, 2]) * t
    m = jnp.maximum(jnp.maximum(l0, l1), l2)
    e0 = jnp.exp(l0 - m)
    e1 = jnp.exp(l1 - m)
    e2 = jnp.exp(l2 - m)
    return 3.0 * (e0 * ol + e1 * oh + e2 * om) / (e0 + e1 + e2)


def _inv_deg(d0_ref, d1_ref):
    d = d0_ref[...] + d1_ref[...]
    return 1.0 / jnp.where(d == 0.0, 1.0, d)


def _fuse1_body(sl_ref, sh_ref, hh_ref, hm_ref, d0_ref, d1_ref, avl_ref,
                avh_ref, avm_ref, attv_ref, wl2_ref, wh2_ref, wm2_ref,
                hl2_ref, hh2_ref, hm2_ref, inv_ref):
    inv = _inv_deg(d0_ref, d1_ref)
    inv_ref[...] = inv
    comb = _fuse_core(sl_ref[...], sh_ref[...], hh_ref[...], hm_ref[...],
                      inv, avl_ref[...], avh_ref[...], avm_ref[...], attv_ref)
    f = jnp.maximum(comb, 0.0)
    hl2_ref[...] = jnp.dot(f, wl2_ref[...], preferred_element_type=jnp.float32)
    hh2_ref[...] = jnp.dot(f, wh2_ref[...], preferred_element_type=jnp.float32)
    hm2_ref[...] = jnp.dot(f, wm2_ref[...], preferred_element_type=jnp.float32)


def _fuse2_body(sl_ref, sh_ref, hh_ref, hm_ref, inv_ref, avl_ref, avh_ref,
                avm_ref, attv_ref, out_ref):
    out_ref[...] = _fuse_core(sl_ref[...], sh_ref[...], hh_ref[...],
                              hm_ref[...], inv_ref[...], avl_ref[...],
                              avh_ref[...], avm_ref[...], attv_ref)


def _tc_fuse1(sl, sh, hh, hm, d0, d1, avl, avh, avm, attv, wl2, wh2, wm2):
    blk = pl.BlockSpec((_BLK, D), lambda i: (i, 0))
    avspec = pl.BlockSpec((1, D), lambda i: (0, 0))
    attspec = pl.BlockSpec(memory_space=pltpu.SMEM)
    wspec = pl.BlockSpec((D, D), lambda i: (0, 0))
    return pl.pallas_call(
        _fuse1_body,
        grid=(_GRID,),
        in_specs=[blk, blk, blk, blk, blk, blk, avspec, avspec, avspec,
                  attspec, wspec, wspec, wspec],
        out_specs=[blk, blk, blk, blk],
        out_shape=[jax.ShapeDtypeStruct((N, D), jnp.float32)] * 4,
    )(sl, sh, hh, hm, d0, d1, avl, avh, avm, attv, wl2, wh2, wm2)


def _tc_fuse2(sl, sh, hh, hm, inv, avl, avh, avm, attv):
    blk = pl.BlockSpec((_BLK, D), lambda i: (i, 0))
    avspec = pl.BlockSpec((1, D), lambda i: (0, 0))
    attspec = pl.BlockSpec(memory_space=pltpu.SMEM)
    return pl.pallas_call(
        _fuse2_body,
        grid=(_GRID,),
        in_specs=[blk, blk, blk, blk, blk, avspec, avspec, avspec, attspec],
        out_specs=blk,
        out_shape=jax.ShapeDtypeStruct((N, D), jnp.float32),
    )(sl, sh, hh, hm, inv, avl, avh, avm, attv)


def kernel(x, edge_index, W_low1, W_high1, W_mlp1, av_low1, av_high1, av_mlp1,
           attv1, W_low2, W_high2, W_mlp2, av_low2, av_high2, av_mlp2, attv2):
    row = edge_index[0]
    col = edge_index[1]

    h1l, h1h, h1m = _tc_matmul3(x, W_low1, W_high1, W_mlp1)
    s1l, s1h, d0, d1 = _sc_spmm_deg(h1l, h1h, row, col)
    h2l, h2h, h2m, inv = _tc_fuse1(
        s1l, s1h, h1h, h1m, d0, d1,
        av_low1.reshape(1, D), av_high1.reshape(1, D), av_mlp1.reshape(1, D),
        attv1, W_low2, W_high2, W_mlp2)
    s2l, s2h = _sc_spmm(h2l, h2h, row, col)
    return _tc_fuse2(
        s2l, s2h, h2h, h2m, inv,
        av_low2.reshape(1, D), av_high2.reshape(1, D), av_mlp2.reshape(1, D),
        attv2)


# speedup vs baseline: 10.5275x; 1.6514x over previous
import functools

import jax
import jax.numpy as jnp
from jax import lax
from jax.experimental import pallas as pl
from jax.experimental.pallas import tpu as pltpu
from jax.experimental.pallas import tpu_sc as plsc

N = 10000
D = 128
E = 320000
NC = 2
NS = 16
NW = NC * NS
ZCHUNK = 80
NZ = N // ZCHUNK
ZITER = (NZ + NS - 1) // NS
SUB = 80
NB = 8
NBATCH = E // (SUB * NB)

_mesh = plsc.VectorSubcoreMesh(
    core_axis_name="c", subcore_axis_name="s", num_cores=NC, num_subcores=NS
)


def _make_sc_spmm(with_deg):
    outs = [
        jax.ShapeDtypeStruct((N, D), jnp.float32),
        jax.ShapeDtypeStruct((N, D), jnp.float32),
    ]
    if with_deg:
        outs += [
            jax.ShapeDtypeStruct((N, D), jnp.float32),
            jax.ShapeDtypeStruct((N, D), jnp.float32),
        ]

    def body(hl, hh, row_hbm, col_hbm, *rest):
        if with_deg:
            out_l, out_h, deg0, deg1 = rest[:4]
            ridx, cidx, rows_a, rows_b, acc, gsa, gsb, ssa, ssb = rest[4:]
        else:
            out_l, out_h = rest[:2]
            ridx, cidx, rows_a, rows_b, acc, gsa, gsb, ssa, ssb = rest[2:]

        c = lax.axis_index("c")
        s = lax.axis_index("s")
        bufs = (rows_a, rows_b)
        gsems = (gsa, gsb)
        ssems = (ssa, ssb)

        z16 = jnp.zeros((16,), jnp.float32)
        one16 = jnp.full((16,), 1.0, jnp.float32)

        def fill(buf, v16):
            def frow(r, carry):
                for k in range(D // 16):
                    buf[r, pl.ds(k * 16, 16)] = v16
                return carry
            lax.fori_loop(0, SUB, frow, 0)

        def drain(sem):
            pltpu.make_async_copy(
                hl.at[pl.ds(0, SUB)], rows_a, sem).wait()

        def zero_acc(zsrc):
            for k in range(ZITER):
                cid = s + NS * k
                off = pl.multiple_of(cid * ZCHUNK, ZCHUNK)

                @pl.when(cid < NZ)
                def _(off=off):
                    pltpu.sync_copy(zsrc, acc.at[pl.ds(off, ZCHUNK)])

        def copy_out(dst):
            for k in range(ZITER):
                cid = s + NS * k
                off = pl.multiple_of(cid * ZCHUNK, ZCHUNK)

                @pl.when(cid < NZ)
                def _(off=off):
                    pltpu.sync_copy(
                        acc.at[pl.ds(off, ZCHUNK)],
                        rows_a.at[pl.ds(0, ZCHUNK)])
                    pltpu.sync_copy(
                        rows_a.at[pl.ds(0, ZCHUNK)],
                        dst.at[pl.ds(off, ZCHUNK)])

        fill(rows_a, z16)
        zero_acc(rows_a)
        plsc.subcore_barrier()

        def run(h_src):
            def batch_body(k, carry):
                bid = s + NS * k

                @pl.when(bid < NBATCH)
                def _():
                    boff = pl.multiple_of(bid * NB, NB)

                    @pl.when(k > 0)
                    def _():
                        drain(ssa)
                        drain(ssb)
                    pltpu.sync_copy(row_hbm.at[pl.ds(boff, NB)], ridx)
                    pltpu.sync_copy(col_hbm.at[pl.ds(boff, NB)], cidx)
                    pltpu.async_copy(
                        h_src.at[cidx.at[0]], bufs[0], gsems[0])
                    for j in range(NB):
                        p = j % 2
                        drain(gsems[p])
                        if j + 1 < NB:
                            if j >= 1:
                                drain(ssems[(j + 1) % 2])
                            pltpu.async_copy(
                                h_src.at[cidx.at[j + 1]],
                                bufs[(j + 1) % 2],
                                gsems[(j + 1) % 2])
                        pltpu.async_copy(
                            bufs[p], acc.at[ridx.at[j]], ssems[p], add=True)
                return carry
            lax.fori_loop(0, (NBATCH + NS - 1) // NS, batch_body, 0)
            drain(ssa)
            drain(ssb)

        @pl.when(c == 0)
        def _():
            run(hl)

        @pl.when(c == 1)
        def _():
            run(hh)

        plsc.subcore_barrier()

        @pl.when(c == 0)
        def _():
            copy_out(out_l)

        @pl.when(c == 1)
        def _():
            copy_out(out_h)

        if with_deg:
            plsc.subcore_barrier()
            fill(rows_b, z16)
            zero_acc(rows_b)
            fill(rows_a, one16)
            plsc.subcore_barrier()

            w = c * NS + s

            def deg_body(k, carry):
                bid = w + NW * k

                @pl.when(bid < NBATCH)
                def _():
                    boff = pl.multiple_of(bid * NB, NB)

                    @pl.when(k > 0)
                    def _():
                        for _j in range(NB):
                            drain(ssa)
                    pltpu.sync_copy(row_hbm.at[pl.ds(boff, NB)], ridx)
                    for j in range(NB):
                        pltpu.async_copy(
                            rows_a, acc.at[ridx.at[j]], ssa, add=True)
                return carry
            lax.fori_loop(0, (NBATCH + NW - 1) // NW, deg_body, 0)
            for _j in range(NB):
                drain(ssa)

            plsc.subcore_barrier()

            @pl.when(c == 0)
            def _():
                copy_out(deg0)

            @pl.when(c == 1)
            def _():
                copy_out(deg1)

    return pl.kernel(
        body,
        out_type=outs,
        mesh=_mesh,
        scratch_types=[
            pltpu.VMEM((NB, SUB), jnp.int32),
            pltpu.VMEM((NB, SUB), jnp.int32),
            pltpu.VMEM((SUB, D), jnp.float32),
            pltpu.VMEM((SUB, D), jnp.float32),
            pltpu.VMEM_SHARED((N, D), jnp.float32),
            pltpu.SemaphoreType.DMA,
            pltpu.SemaphoreType.DMA,
            pltpu.SemaphoreType.DMA,
            pltpu.SemaphoreType.DMA,
        ],
        name="sc_spmm_deg" if with_deg else "sc_spmm",
    )


_sc_spmm_deg = _make_sc_spmm(True)
_sc_spmm = _make_sc_spmm(False)


_BLK = 1000
_GRID = N // _BLK


def _mm3_body(x_ref, wl_ref, wh_ref, wm_ref, hl_ref, hh_ref, hm_ref):
    xb = x_ref[...]
    hl_ref[...] = jnp.dot(xb, wl_ref[...], preferred_element_type=jnp.float32)
    hh_ref[...] = jnp.dot(xb, wh_ref[...], preferred_element_type=jnp.float32)
    hm_ref[...] = jnp.dot(xb, wm_ref[...], preferred_element_type=jnp.float32)


def _tc_matmul3(x, wl, wh, wm):
    blk = pl.BlockSpec((_BLK, D), lambda i: (i, 0))
    wspec = pl.BlockSpec((D, D), lambda i: (0, 0))
    return pl.pallas_call(
        _mm3_body,
        grid=(_GRID,),
        in_specs=[blk, wspec, wspec, wspec],
        out_specs=[blk, blk, blk],
        out_shape=[jax.ShapeDtypeStruct((N, D), jnp.float32)] * 3,
    )(x, wl, wh, wm)


def _fuse_core(sl, sh, hh, hm, inv, avl, avh, avm, attv_ref):
    ol = jnp.maximum(sl * inv, 0.0)
    oh = jnp.maximum(hh - sh * inv, 0.0)
    om = jnp.maximum(hm, 0.0)
    z0 = jax.nn.sigmoid(jnp.sum(ol * avl, axis=1, keepdims=True))
    z1 = jax.nn.sigmoid(jnp.sum(oh * avh, axis=1, keepdims=True))
    z2 = jax.nn.sigmoid(jnp.sum(om * avm, axis=1, keepdims=True))
    t = 1.0 / 3.0
    l0 = (z0 * attv_ref[0, 0] + z1 * attv_ref[1, 0] + z2 * attv_ref[2, 0]) * t
    l1 = (z0 * attv_ref[0, 1] + z1 * attv_ref[1, 1] + z2 * attv_ref[2, 1]) * t
    l2 = (z0 * attv_ref[0, 2] + z1 * attv_ref[1, 2] + z2 * attv_ref[2, 2]) * t
    m = jnp.maximum(jnp.maximum(l0, l1), l2)
    e0 = jnp.exp(l0 - m)
    e1 = jnp.exp(l1 - m)
    e2 = jnp.exp(l2 - m)
    return 3.0 * (e0 * ol + e1 * oh + e2 * om) / (e0 + e1 + e2)


def _inv_deg(d0_ref, d1_ref):
    d = d0_ref[...] + d1_ref[...]
    return 1.0 / jnp.where(d == 0.0, 1.0, d)


def _fuse1_body(sl_ref, sh_ref, hh_ref, hm_ref, d0_ref, d1_ref, avl_ref,
                avh_ref, avm_ref, attv_ref, wl2_ref, wh2_ref, wm2_ref,
                hl2_ref, hh2_ref, hm2_ref, inv_ref):
    inv = _inv_deg(d0_ref, d1_ref)
    inv_ref[...] = inv
    comb = _fuse_core(sl_ref[...], sh_ref[...], hh_ref[...], hm_ref[...],
                      inv, avl_ref[...], avh_ref[...], avm_ref[...], attv_ref)
    f = jnp.maximum(comb, 0.0)
    hl2_ref[...] = jnp.dot(f, wl2_ref[...], preferred_element_type=jnp.float32)
    hh2_ref[...] = jnp.dot(f, wh2_ref[...], preferred_element_type=jnp.float32)
    hm2_ref[...] = jnp.dot(f, wm2_ref[...], preferred_element_type=jnp.float32)


def _fuse2_body(sl_ref, sh_ref, hh_ref, hm_ref, inv_ref, avl_ref, avh_ref,
                avm_ref, attv_ref, out_ref):
    out_ref[...] = _fuse_core(sl_ref[...], sh_ref[...], hh_ref[...],
                              hm_ref[...], inv_ref[...], avl_ref[...],
                              avh_ref[...], avm_ref[...], attv_ref)


def _tc_fuse1(sl, sh, hh, hm, d0, d1, avl, avh, avm, attv, wl2, wh2, wm2):
    blk = pl.BlockSpec((_BLK, D), lambda i: (i, 0))
    avspec = pl.BlockSpec((1, D), lambda i: (0, 0))
    attspec = pl.BlockSpec(memory_space=pltpu.SMEM)
    wspec = pl.BlockSpec((D, D), lambda i: (0, 0))
    return pl.pallas_call(
        _fuse1_body,
        grid=(_GRID,),
        in_specs=[blk, blk, blk, blk, blk, blk, avspec, avspec, avspec,
                  attspec, wspec, wspec, wspec],
        out_specs=[blk, blk, blk, blk],
        out_shape=[jax.ShapeDtypeStruct((N, D), jnp.float32)] * 4,
    )(sl, sh, hh, hm, d0, d1, avl, avh, avm, attv, wl2, wh2, wm2)


def _tc_fuse2(sl, sh, hh, hm, inv, avl, avh, avm, attv):
    blk = pl.BlockSpec((_BLK, D), lambda i: (i, 0))
    avspec = pl.BlockSpec((1, D), lambda i: (0, 0))
    attspec = pl.BlockSpec(memory_space=pltpu.SMEM)
    return pl.pallas_call(
        _fuse2_body,
        grid=(_GRID,),
        in_specs=[blk, blk, blk, blk, blk, avspec, avspec, avspec, attspec],
        out_specs=blk,
        out_shape=jax.ShapeDtypeStruct((N, D), jnp.float32),
    )(sl, sh, hh, hm, inv, avl, avh, avm, attv)


def kernel(x, edge_index, W_low1, W_high1, W_mlp1, av_low1, av_high1, av_mlp1,
           attv1, W_low2, W_high2, W_mlp2, av_low2, av_high2, av_mlp2, attv2):
    row = edge_index[0].reshape(E // SUB, SUB)
    col = edge_index[1].reshape(E // SUB, SUB)

    h1l, h1h, h1m = _tc_matmul3(x, W_low1, W_high1, W_mlp1)
    s1l, s1h, d0, d1 = _sc_spmm_deg(h1l, h1h, row, col)
    h2l, h2h, h2m, inv = _tc_fuse1(
        s1l, s1h, h1h, h1m, d0, d1,
        av_low1.reshape(1, D), av_high1.reshape(1, D), av_mlp1.reshape(1, D),
        attv1, W_low2, W_high2, W_mlp2)
    s2l, s2h = _sc_spmm(h2l, h2h, row, col)
    return _tc_fuse2(
        s2l, s2h, h2h, h2m, inv,
        av_low2.reshape(1, D), av_high2.reshape(1, D), av_mlp2.reshape(1, D),
        attv2)
